# Initial kernel scaffold; baseline (speedup 1.0000x reference)
import jax
import jax.numpy as jnp
from jax.experimental import pallas as pl


def kernel(in_features, pair_first, pair_second, dist_pairs, coord_pairs, int_weights, self_W, self_b, vecscales, mu, sigma):
    raise NotImplementedError("write your pallas kernel here")


# speedup vs baseline: 32.9377x; 32.9377x over previous
import functools

import jax
import jax.numpy as jnp
from jax import lax
from jax.experimental import pallas as pl
from jax.experimental.pallas import tpu as pltpu
from jax.experimental.pallas import tpu_sc as plsc

HARD_CUTOFF = 5.5
CUSP_REG = 1e-06
W = 640
TA = 80


def _sc_gather(table, idx):
    E = idx.shape[0]
    nf = table.shape[1]
    info = plsc.get_sparse_core_info()
    nw = info.num_cores * info.num_subcores
    b_per_w = E // nw
    C = 1000
    n_iter = b_per_w // C
    mesh = plsc.VectorSubcoreMesh(core_axis_name="c", subcore_axis_name="s")

    @functools.partial(
        pl.kernel, mesh=mesh,
        out_type=jax.ShapeDtypeStruct((E, nf), jnp.float32),
        scratch_types=[
            pltpu.VMEM((C,), jnp.int32),
            pltpu.VMEM((C, nf), jnp.float32),
            pltpu.SemaphoreType.DMA,
        ],
    )
    def k(table_hbm, idx_hbm, out_hbm, idx_v, rows_v, sem):
        wid = lax.axis_index("s") * info.num_cores + lax.axis_index("c")
        base = wid * b_per_w

        def body(j, carry):
            off = base + j * C
            pltpu.sync_copy(idx_hbm.at[pl.ds(off, C)], idx_v)
            pltpu.async_copy(table_hbm.at[idx_v], rows_v, sem).wait()
            pltpu.sync_copy(rows_v, out_hbm.at[pl.ds(off, C)])
            return carry

        lax.fori_loop(0, n_iter, body, 0)

    return k(table, idx)


def _tc_body(win_ref, blk_ref, feat_ref, pf_ref, ed_ref, wrs_ref, sw_ref,
             par_ref, inf_ref, out_ref, acc_ref):
    u = pl.program_id(0)
    nu = pl.num_programs(0)
    b = blk_ref[u]
    prev_b = blk_ref[jnp.maximum(u - 1, 0)]
    next_b = blk_ref[jnp.minimum(u + 1, nu - 1)]
    first = jnp.logical_or(u == 0, b != prev_b)
    last = jnp.logical_or(u == nu - 1, b != next_b)

    @pl.when(first)
    def _():
        acc_ref[...] = jnp.zeros_like(acc_ref)

    nd = wrs_ref.shape[0] // feat_ref.shape[1]
    ed = ed_ref[0]
    d = ed[:, 0:1]
    inv = 1.0 / d
    cut = jnp.where(d < HARD_CUTOFF,
                    jnp.cos((0.5 * jnp.pi / HARD_CUTOFF) * d) ** 2, 0.0)
    mu = par_ref[0:1, 0:nd]
    sg = par_ref[1:2, 0:nd]
    sense = jnp.exp(-0.5 * ((inv - mu) / sg) ** 2) * cut

    feat = feat_ref[...]
    z = jnp.concatenate([sense[:, c:c + 1] * feat for c in range(nd)], axis=1)
    r = jnp.dot(z, wrs_ref[...], preferred_element_type=jnp.float32)

    ux = ed[:, 1:2] * inv
    uy = ed[:, 2:3] * inv
    uz = ed[:, 3:4] * inv
    r4 = jnp.concatenate([r, ux * r, uy * r, uz * r], axis=1)

    pf = pf_ref[0, 0, :]
    aid = b * TA + lax.broadcasted_iota(jnp.int32, (TA, W), 0)
    oh = (aid == pf[None, :]).astype(jnp.float32)
    acc_ref[...] += jnp.dot(oh, r4, preferred_element_type=jnp.float32)

    @pl.when(last)
    def _():
        a = acc_ref[...]
        nf = out_ref.shape[1]
        fs = a[:, 0:nf]
        fx = a[:, nf:2 * nf]
        fy = a[:, 2 * nf:3 * nf]
        fz = a[:, 3 * nf:4 * nf]
        fv = jnp.sqrt(fx * fx + fy * fy + fz * fz + CUSP_REG) * par_ref[2:3, :]
        sp = jnp.dot(inf_ref[...], sw_ref[...],
                     preferred_element_type=jnp.float32) + par_ref[3:4, :]
        out_ref[...] = fs + fv + sp


def _make_grid_spec(U, nwin_tot, n_atoms, nf, nd):
    def wmap(u, wi, bi):
        return (wi[u], 0)

    def wmap3(u, wi, bi):
        return (wi[u], 0, 0)

    def bmap(u, wi, bi):
        return (bi[u], 0)

    const2 = lambda u, wi, bi: (0, 0)
    return pltpu.PrefetchScalarGridSpec(
        num_scalar_prefetch=2,
        grid=(U,),
        in_specs=[
            pl.BlockSpec((W, nf), wmap),
            pl.BlockSpec((1, 1, W), wmap3),
            pl.BlockSpec((1, W, 8), wmap3),
            pl.BlockSpec((nd * nf, nf), const2),
            pl.BlockSpec((nf, nf), const2),
            pl.BlockSpec((8, 128), const2),
            pl.BlockSpec((TA, nf), bmap),
        ],
        out_specs=pl.BlockSpec((TA, nf), bmap),
        scratch_shapes=[pltpu.VMEM((TA, 4 * nf), jnp.float32)],
    )


def kernel(in_features, pair_first, pair_second, dist_pairs, coord_pairs,
           int_weights, self_W, self_b, vecscales, mu, sigma):
    n_atoms, nf = in_features.shape
    E = pair_first.shape[0]
    nd = mu.shape[0]
    nwin = E // W
    nwin_tot = nwin + 1
    nblk = n_atoms // TA
    U = nwin + nblk

    feat_g = _sc_gather(in_features, pair_second.astype(jnp.int32))

    feat_pad = jnp.concatenate(
        [feat_g, jnp.zeros((W, nf), jnp.float32)], axis=0)
    pf_pad = jnp.concatenate(
        [pair_first, jnp.full((W,), n_atoms, jnp.int32)])
    ed = jnp.concatenate(
        [dist_pairs[:, None], coord_pairs, jnp.zeros((E, 4), jnp.float32)],
        axis=1)
    ed_tail = jnp.tile(jnp.array([[1.0] + [0.0] * 7], jnp.float32), (W, 1))
    ed = jnp.concatenate([ed, ed_tail], axis=0).reshape(nwin_tot, W, 8)
    pf3 = pf_pad.reshape(nwin_tot, 1, W)

    fb = pair_first[::W] // TA
    lb = pair_first[W - 1::W] // TA
    cs = jnp.concatenate(
        [jnp.zeros((1,), lb.dtype), jnp.minimum(fb[1:], lb[:-1] + 1)])
    ce = jnp.concatenate([lb[:-1], jnp.full((1,), nblk - 1, lb.dtype)])
    counts = ce - cs + 1
    off = jnp.concatenate(
        [jnp.zeros((1,), jnp.int32), jnp.cumsum(counts).astype(jnp.int32)])
    u_real = off[-1]
    uu = jnp.arange(U, dtype=jnp.int32)
    w_of = jnp.clip(jnp.searchsorted(off, uu, side="right") - 1, 0, nwin - 1)
    b_of = cs[w_of].astype(jnp.int32) + (uu - off[w_of])
    valid = uu < u_real
    win_ids = jnp.where(valid, w_of, nwin).astype(jnp.int32)
    blk_ids = jnp.where(valid, b_of, nblk - 1).astype(jnp.int32)

    wrs = jnp.transpose(int_weights, (0, 2, 1)).reshape(nd * nf, -1)
    sw_t = self_W.T
    params = jnp.zeros((8, 128), jnp.float32)
    params = params.at[0, :nd].set(mu)
    params = params.at[1, :nd].set(sigma)
    params = params.at[2, :].set(vecscales)
    params = params.at[3, :].set(self_b)

    out = pl.pallas_call(
        _tc_body,
        grid_spec=_make_grid_spec(U, nwin_tot, n_atoms, nf, nd),
        out_shape=jax.ShapeDtypeStruct((n_atoms, nf), jnp.float32),
        compiler_params=pltpu.CompilerParams(
            dimension_semantics=("arbitrary",)),
    )(win_ids, blk_ids, feat_pad, pf3, ed, wrs, sw_t, params, in_features)
    return out


# speedup vs baseline: 50.8174x; 1.5428x over previous
import functools

import jax
import jax.numpy as jnp
from jax import lax
from jax.experimental import pallas as pl
from jax.experimental.pallas import tpu as pltpu
from jax.experimental.pallas import tpu_sc as plsc

HARD_CUTOFF = 5.5
CUSP_REG = 1e-06
W = 640
TA = 80


def _sc_gather(table, idx):
    E = idx.shape[0]
    nf = table.shape[1]
    info = plsc.get_sparse_core_info()
    nw = info.num_cores * info.num_subcores
    b_per_w = E // nw
    C = 1000
    n_iter = b_per_w // C
    mesh = plsc.VectorSubcoreMesh(core_axis_name="c", subcore_axis_name="s")

    @functools.partial(
        pl.kernel, mesh=mesh,
        out_type=jax.ShapeDtypeStruct((E, nf), jnp.float32),
        scratch_types=[
            pltpu.VMEM((C,), jnp.int32),
            pltpu.VMEM((C, nf), jnp.float32),
            pltpu.SemaphoreType.DMA,
        ],
    )
    def k(table_hbm, idx_hbm, out_hbm, idx_v, rows_v, sem):
        wid = lax.axis_index("s") * info.num_cores + lax.axis_index("c")
        base = wid * b_per_w

        def body(j, carry):
            off = base + j * C
            pltpu.sync_copy(idx_hbm.at[pl.ds(off, C)], idx_v)
            pltpu.async_copy(table_hbm.at[idx_v], rows_v, sem).wait()
            pltpu.sync_copy(rows_v, out_hbm.at[pl.ds(off, C)])
            return carry

        lax.fori_loop(0, n_iter, body, 0)

    return k(table, idx)


def _tc_body(win_ref, blk_ref, feat_ref, pf_ref, ed_ref, wrs_ref, sw_ref,
             par_ref, inf_ref, out_ref, acc_ref, r4_ref):
    u = pl.program_id(0)
    nu = pl.num_programs(0)
    b = blk_ref[u]
    prev_b = blk_ref[jnp.maximum(u - 1, 0)]
    next_b = blk_ref[jnp.minimum(u + 1, nu - 1)]
    first = jnp.logical_or(u == 0, b != prev_b)
    last = jnp.logical_or(u == nu - 1, b != next_b)
    new_win = jnp.logical_or(
        u == 0, win_ref[u] != win_ref[jnp.maximum(u - 1, 0)])

    @pl.when(first)
    def _():
        acc_ref[...] = jnp.zeros_like(acc_ref)

    @pl.when(new_win)
    def _():
        nd = wrs_ref.shape[0] // feat_ref.shape[1]
        ed = ed_ref[0]
        d = ed[:, 0:1]
        inv = 1.0 / d
        cut = jnp.where(d < HARD_CUTOFF,
                        jnp.cos((0.5 * jnp.pi / HARD_CUTOFF) * d) ** 2, 0.0)
        mu = par_ref[0:1, 0:nd]
        sg = par_ref[1:2, 0:nd]
        sense = (jnp.exp(-0.5 * ((inv - mu) / sg) ** 2)
                 * cut).astype(jnp.bfloat16)

        feat = feat_ref[...].astype(jnp.bfloat16)
        z = jnp.concatenate(
            [sense[:, c:c + 1] * feat for c in range(nd)], axis=1)
        r = jnp.dot(z, wrs_ref[...],
                    preferred_element_type=jnp.float32)

        ux = ed[:, 1:2] * inv
        uy = ed[:, 2:3] * inv
        uz = ed[:, 3:4] * inv
        r4 = jnp.concatenate([r, ux * r, uy * r, uz * r], axis=1)
        r4_ref[...] = r4.astype(jnp.bfloat16)

    pf = pf_ref[0, 0, :]
    aid = b * TA + lax.broadcasted_iota(jnp.int32, (TA, W), 0)
    oh = (aid == pf[None, :]).astype(jnp.bfloat16)
    acc_ref[...] += jnp.dot(oh, r4_ref[...],
                            preferred_element_type=jnp.float32)

    @pl.when(last)
    def _():
        a = acc_ref[...]
        nf = out_ref.shape[1]
        fs = a[:, 0:nf]
        fx = a[:, nf:2 * nf]
        fy = a[:, 2 * nf:3 * nf]
        fz = a[:, 3 * nf:4 * nf]
        fv = jnp.sqrt(fx * fx + fy * fy + fz * fz + CUSP_REG) * par_ref[2:3, :]
        sp = jnp.dot(inf_ref[...], sw_ref[...],
                     preferred_element_type=jnp.float32) + par_ref[3:4, :]
        out_ref[...] = fs + fv + sp


def _make_grid_spec(U, nwin_tot, n_atoms, nf, nd):
    nwin = nwin_tot - 1

    def wmapf(u, wi, bi):
        return (jnp.minimum(wi[u], nwin - 1), 0)

    def wmap3(u, wi, bi):
        return (wi[u], 0, 0)

    def bmap(u, wi, bi):
        return (bi[u], 0)

    const2 = lambda u, wi, bi: (0, 0)
    return pltpu.PrefetchScalarGridSpec(
        num_scalar_prefetch=2,
        grid=(U,),
        in_specs=[
            pl.BlockSpec((W, nf), wmapf),
            pl.BlockSpec((1, 1, W), wmap3),
            pl.BlockSpec((1, W, 8), wmap3),
            pl.BlockSpec((nd * nf, nf), const2),
            pl.BlockSpec((nf, nf), const2),
            pl.BlockSpec((8, 128), const2),
            pl.BlockSpec((TA, nf), bmap),
        ],
        out_specs=pl.BlockSpec((TA, nf), bmap),
        scratch_shapes=[pltpu.VMEM((TA, 4 * nf), jnp.float32),
                        pltpu.VMEM((W, 4 * nf), jnp.bfloat16)],
    )


def kernel(in_features, pair_first, pair_second, dist_pairs, coord_pairs,
           int_weights, self_W, self_b, vecscales, mu, sigma):
    n_atoms, nf = in_features.shape
    E = pair_first.shape[0]
    nd = mu.shape[0]
    nwin = E // W
    nwin_tot = nwin + 1
    nblk = n_atoms // TA
    U = nwin + nblk

    feat_g = _sc_gather(in_features, pair_second.astype(jnp.int32))

    pf_pad = jnp.concatenate(
        [pair_first, jnp.full((W,), n_atoms, jnp.int32)])
    ed = jnp.concatenate(
        [dist_pairs[:, None], coord_pairs, jnp.zeros((E, 4), jnp.float32)],
        axis=1)
    ed_tail = jnp.tile(jnp.array([[1.0] + [0.0] * 7], jnp.float32), (W, 1))
    ed = jnp.concatenate([ed, ed_tail], axis=0).reshape(nwin_tot, W, 8)
    pf3 = pf_pad.reshape(nwin_tot, 1, W)

    fb = pair_first[::W] // TA
    lb = pair_first[W - 1::W] // TA
    cs = jnp.concatenate(
        [jnp.zeros((1,), lb.dtype), jnp.minimum(fb[1:], lb[:-1] + 1)])
    ce = jnp.concatenate([lb[:-1], jnp.full((1,), nblk - 1, lb.dtype)])
    counts = ce - cs + 1
    off = jnp.concatenate(
        [jnp.zeros((1,), jnp.int32), jnp.cumsum(counts).astype(jnp.int32)])
    u_real = off[-1]
    uu = jnp.arange(U, dtype=jnp.int32)
    w_of = jnp.clip(jnp.searchsorted(off, uu, side="right") - 1, 0, nwin - 1)
    b_of = cs[w_of].astype(jnp.int32) + (uu - off[w_of])
    valid = uu < u_real
    win_ids = jnp.where(valid, w_of, nwin).astype(jnp.int32)
    blk_ids = jnp.where(valid, b_of, nblk - 1).astype(jnp.int32)

    wrs = jnp.transpose(int_weights, (0, 2, 1)).reshape(
        nd * nf, -1).astype(jnp.bfloat16)
    sw_t = self_W.T
    params = jnp.zeros((8, 128), jnp.float32)
    params = params.at[0, :nd].set(mu)
    params = params.at[1, :nd].set(sigma)
    params = params.at[2, :].set(vecscales)
    params = params.at[3, :].set(self_b)

    out = pl.pallas_call(
        _tc_body,
        grid_spec=_make_grid_spec(U, nwin_tot, n_atoms, nf, nd),
        out_shape=jax.ShapeDtypeStruct((n_atoms, nf), jnp.float32),
        compiler_params=pltpu.CompilerParams(
            dimension_semantics=("arbitrary",)),
    )(win_ids, blk_ids, feat_g, pf3, ed, wrs, sw_t, params, in_features)
    return out


# speedup vs baseline: 64.0868x; 1.2611x over previous
import functools

import jax
import jax.numpy as jnp
from jax import lax
from jax.experimental import pallas as pl
from jax.experimental.pallas import tpu as pltpu
from jax.experimental.pallas import tpu_sc as plsc

HARD_CUTOFF = 5.5
CUSP_REG = 1e-06
W = 640
TA = 80


def _sc_gather(table, idx):
    E = idx.shape[0]
    nf = table.shape[1]
    info = plsc.get_sparse_core_info()
    nw = info.num_cores * info.num_subcores
    b_per_w = E // nw
    C = 1000
    n_iter = b_per_w // C
    mesh = plsc.VectorSubcoreMesh(core_axis_name="c", subcore_axis_name="s")

    @functools.partial(
        pl.kernel, mesh=mesh,
        out_type=jax.ShapeDtypeStruct((E, nf), jnp.float32),
        scratch_types=[
            pltpu.VMEM((C,), jnp.int32),
            pltpu.VMEM((C, nf), jnp.float32),
            pltpu.SemaphoreType.DMA,
        ],
    )
    def k(table_hbm, idx_hbm, out_hbm, idx_v, rows_v, sem):
        wid = lax.axis_index("s") * info.num_cores + lax.axis_index("c")
        base = wid * b_per_w

        def body(j, carry):
            off = base + j * C
            pltpu.sync_copy(idx_hbm.at[pl.ds(off, C)], idx_v)
            pltpu.async_copy(table_hbm.at[idx_v], rows_v, sem).wait()
            pltpu.sync_copy(rows_v, out_hbm.at[pl.ds(off, C)])
            return carry

        lax.fori_loop(0, n_iter, body, 0)

    return k(table, idx)


def _tc_body(win_ref, blk_ref, feat_ref, pf_ref, ed_ref, wrs_ref, sw_ref,
             par_ref, inf_ref, out_ref, acc_ref, r4_ref):
    u = pl.program_id(0)
    nu = pl.num_programs(0)
    b = blk_ref[u]
    prev_b = blk_ref[jnp.maximum(u - 1, 0)]
    next_b = blk_ref[jnp.minimum(u + 1, nu - 1)]
    first = jnp.logical_or(u == 0, b != prev_b)
    last = jnp.logical_or(u == nu - 1, b != next_b)
    new_win = jnp.logical_or(
        u == 0, win_ref[u] != win_ref[jnp.maximum(u - 1, 0)])

    @pl.when(first)
    def _():
        acc_ref[...] = jnp.zeros_like(acc_ref)

    @pl.when(new_win)
    def _():
        nd = wrs_ref.shape[0] // feat_ref.shape[1]
        edt = ed_ref[0]
        dr = edt[0:1, :]
        invr = 1.0 / dr
        cutr = jnp.where(dr < HARD_CUTOFF,
                         jnp.cos((0.5 * jnp.pi / HARD_CUTOFF) * dr) ** 2, 0.0)
        ur = edt[1:4, :] * invr
        scal = jnp.concatenate(
            [invr, cutr, ur, jnp.zeros((3, W), jnp.float32)], axis=0)
        cols = scal.T
        inv = cols[:, 0:1]
        cut = cols[:, 1:2]
        mu = par_ref[0:1, 0:nd]
        sg = par_ref[1:2, 0:nd]
        sense = (jnp.exp(-0.5 * ((inv - mu) / sg) ** 2)
                 * cut).astype(jnp.bfloat16)

        feat = feat_ref[...].astype(jnp.bfloat16)
        z = jnp.concatenate(
            [sense[:, c:c + 1] * feat for c in range(nd)], axis=1)
        r = jnp.dot(z, wrs_ref[...],
                    preferred_element_type=jnp.float32)

        ux = cols[:, 2:3]
        uy = cols[:, 3:4]
        uz = cols[:, 4:5]
        r4 = jnp.concatenate([r, ux * r, uy * r, uz * r], axis=1)
        r4_ref[...] = r4.astype(jnp.bfloat16)

    pf = pf_ref[0, 0, :]
    aid = b * TA + lax.broadcasted_iota(jnp.int32, (TA, W), 0)
    oh = (aid == pf[None, :]).astype(jnp.bfloat16)
    acc_ref[...] += jnp.dot(oh, r4_ref[...],
                            preferred_element_type=jnp.float32)

    @pl.when(last)
    def _():
        a = acc_ref[...]
        nf = out_ref.shape[1]
        fs = a[:, 0:nf]
        fx = a[:, nf:2 * nf]
        fy = a[:, 2 * nf:3 * nf]
        fz = a[:, 3 * nf:4 * nf]
        fv = jnp.sqrt(fx * fx + fy * fy + fz * fz + CUSP_REG) * par_ref[2:3, :]
        sp = jnp.dot(inf_ref[...], sw_ref[...],
                     preferred_element_type=jnp.float32) + par_ref[3:4, :]
        out_ref[...] = fs + fv + sp


def _make_grid_spec(U, nwin_tot, n_atoms, nf, nd):
    nwin = nwin_tot - 1

    def wmapf(u, wi, bi):
        return (jnp.minimum(wi[u], nwin - 1), 0)

    def wmap3(u, wi, bi):
        return (wi[u], 0, 0)

    def bmap(u, wi, bi):
        return (bi[u], 0)

    const2 = lambda u, wi, bi: (0, 0)
    return pltpu.PrefetchScalarGridSpec(
        num_scalar_prefetch=2,
        grid=(U,),
        in_specs=[
            pl.BlockSpec((W, nf), wmapf),
            pl.BlockSpec((1, 1, W), wmap3),
            pl.BlockSpec((1, 8, W), wmap3),
            pl.BlockSpec((nd * nf, nf), const2),
            pl.BlockSpec((nf, nf), const2),
            pl.BlockSpec((8, 128), const2),
            pl.BlockSpec((TA, nf), bmap),
        ],
        out_specs=pl.BlockSpec((TA, nf), bmap),
        scratch_shapes=[pltpu.VMEM((TA, 4 * nf), jnp.float32),
                        pltpu.VMEM((W, 4 * nf), jnp.bfloat16)],
    )


def kernel(in_features, pair_first, pair_second, dist_pairs, coord_pairs,
           int_weights, self_W, self_b, vecscales, mu, sigma):
    n_atoms, nf = in_features.shape
    E = pair_first.shape[0]
    nd = mu.shape[0]
    nwin = E // W
    nwin_tot = nwin + 1
    nblk = n_atoms // TA
    U = nwin + nblk

    feat_g = _sc_gather(in_features, pair_second.astype(jnp.int32))

    pf_pad = jnp.concatenate(
        [pair_first, jnp.full((W,), n_atoms, jnp.int32)])
    ed = jnp.concatenate(
        [dist_pairs[:, None], coord_pairs, jnp.zeros((E, 4), jnp.float32)],
        axis=1)
    ed_tail = jnp.tile(jnp.array([[1.0] + [0.0] * 7], jnp.float32), (W, 1))
    ed = jnp.concatenate([ed, ed_tail], axis=0).reshape(
        nwin_tot, W, 8).transpose(0, 2, 1)
    pf3 = pf_pad.reshape(nwin_tot, 1, W)

    fb = pair_first[::W] // TA
    lb = pair_first[W - 1::W] // TA
    cs = jnp.concatenate(
        [jnp.zeros((1,), lb.dtype), jnp.minimum(fb[1:], lb[:-1] + 1)])
    ce = jnp.concatenate([lb[:-1], jnp.full((1,), nblk - 1, lb.dtype)])
    counts = ce - cs + 1
    off = jnp.concatenate(
        [jnp.zeros((1,), jnp.int32), jnp.cumsum(counts).astype(jnp.int32)])
    u_real = off[-1]
    uu = jnp.arange(U, dtype=jnp.int32)
    w_of = jnp.clip(jnp.searchsorted(off, uu, side="right") - 1, 0, nwin - 1)
    b_of = cs[w_of].astype(jnp.int32) + (uu - off[w_of])
    valid = uu < u_real
    win_ids = jnp.where(valid, w_of, nwin).astype(jnp.int32)
    blk_ids = jnp.where(valid, b_of, nblk - 1).astype(jnp.int32)

    wrs = jnp.transpose(int_weights, (0, 2, 1)).reshape(
        nd * nf, -1).astype(jnp.bfloat16)
    sw_t = self_W.T
    params = jnp.zeros((8, 128), jnp.float32)
    params = params.at[0, :nd].set(mu)
    params = params.at[1, :nd].set(sigma)
    params = params.at[2, :].set(vecscales)
    params = params.at[3, :].set(self_b)

    out = pl.pallas_call(
        _tc_body,
        grid_spec=_make_grid_spec(U, nwin_tot, n_atoms, nf, nd),
        out_shape=jax.ShapeDtypeStruct((n_atoms, nf), jnp.float32),
        compiler_params=pltpu.CompilerParams(
            dimension_semantics=("arbitrary",)),
    )(win_ids, blk_ids, feat_g, pf3, ed, wrs, sw_t, params, in_features)
    return out


# speedup vs baseline: 68.8347x; 1.0741x over previous
import functools

import jax
import jax.numpy as jnp
from jax import lax
from jax.experimental import pallas as pl
from jax.experimental.pallas import tpu as pltpu
from jax.experimental.pallas import tpu_sc as plsc

HARD_CUTOFF = 5.5
CUSP_REG = 1e-06
W = 640
TA = 80


def _sc_gather(table, idx):
    E = idx.shape[0]
    nf = table.shape[1]
    info = plsc.get_sparse_core_info()
    nw = info.num_cores * info.num_subcores
    b_per_w = E // nw
    C = 1000
    n_iter = b_per_w // C
    mesh = plsc.VectorSubcoreMesh(core_axis_name="c", subcore_axis_name="s")

    @functools.partial(
        pl.kernel, mesh=mesh,
        out_type=jax.ShapeDtypeStruct((E, nf), jnp.float32),
        scratch_types=[
            pltpu.VMEM((C,), jnp.int32),
            pltpu.VMEM((C, nf), jnp.float32),
            pltpu.SemaphoreType.DMA,
        ],
    )
    def k(table_hbm, idx_hbm, out_hbm, idx_v, rows_v, sem):
        wid = lax.axis_index("s") * info.num_cores + lax.axis_index("c")
        base = wid * b_per_w

        def body(j, carry):
            off = base + j * C
            pltpu.sync_copy(idx_hbm.at[pl.ds(off, C)], idx_v)
            pltpu.async_copy(table_hbm.at[idx_v], rows_v, sem).wait()
            pltpu.sync_copy(rows_v, out_hbm.at[pl.ds(off, C)])
            return carry

        lax.fori_loop(0, n_iter, body, 0)

    return k(table, idx)


def _tc_body(win_ref, blk_ref, live_ref, feat_ref, pf_ref, ed_ref, wrs_ref,
             sw_ref, par_ref, inf_ref, out_ref, acc_ref, r4_ref):
    u = pl.program_id(0)
    nu = pl.num_programs(0)
    b = blk_ref[u]
    prev_b = blk_ref[jnp.maximum(u - 1, 0)]
    next_b = blk_ref[jnp.minimum(u + 1, nu - 1)]
    first = jnp.logical_or(u == 0, b != prev_b)
    last = jnp.logical_or(u == nu - 1, b != next_b)
    live = live_ref[u] > 0
    new_win = jnp.logical_or(
        u == 0, win_ref[u] != win_ref[jnp.maximum(u - 1, 0)])

    @pl.when(first)
    def _():
        acc_ref[...] = jnp.zeros_like(acc_ref)

    @pl.when(jnp.logical_and(live, new_win))
    def _():
        nf = feat_ref.shape[1]
        nd = wrs_ref.shape[0] // nf
        edt = ed_ref[0]
        dr = edt[0:1, :]
        invr = 1.0 / dr
        cutr = jnp.where(dr < HARD_CUTOFF,
                         jnp.cos((0.5 * jnp.pi / HARD_CUTOFF) * dr) ** 2, 0.0)
        ur = edt[1:4, :] * invr
        muc = par_ref[8:8 + nd, 0:1]
        isg = par_ref[8:8 + nd, 1:2]
        st = jnp.exp(-0.5 * ((invr - muc) * isg) ** 2) * cutr
        sense = st.T.astype(jnp.bfloat16)

        feat = feat_ref[...].astype(jnp.bfloat16)
        z = jnp.concatenate(
            [sense[:, c:c + 1] * feat for c in range(nd)], axis=1)
        r = jnp.dot(z, wrs_ref[...],
                    preferred_element_type=jnp.float32)
        rb = r.astype(jnp.bfloat16)

        scal = jnp.concatenate(
            [ur, jnp.zeros((5, W), jnp.float32)], axis=0)
        cols = scal.T.astype(jnp.bfloat16)
        r4_ref[:, 0:nf] = rb
        r4_ref[:, nf:2 * nf] = cols[:, 0:1] * rb
        r4_ref[:, 2 * nf:3 * nf] = cols[:, 1:2] * rb
        r4_ref[:, 3 * nf:4 * nf] = cols[:, 2:3] * rb

    @pl.when(live)
    def _():
        pf = pf_ref[0, 0, :]
        aid = b * TA + lax.broadcasted_iota(jnp.int32, (TA, W), 0)
        oh = (aid == pf[None, :]).astype(jnp.bfloat16)
        acc_ref[...] += jnp.dot(oh, r4_ref[...],
                                preferred_element_type=jnp.float32)

    @pl.when(last)
    def _():
        a = acc_ref[...]
        nf = out_ref.shape[1]
        fs = a[:, 0:nf]
        fx = a[:, nf:2 * nf]
        fy = a[:, 2 * nf:3 * nf]
        fz = a[:, 3 * nf:4 * nf]
        fv = jnp.sqrt(fx * fx + fy * fy + fz * fz + CUSP_REG) * par_ref[0:1, :]
        sp = jnp.dot(inf_ref[...], sw_ref[...],
                     preferred_element_type=jnp.float32) + par_ref[1:2, :]
        out_ref[...] = fs + fv + sp


def _make_grid_spec(U, nwin_tot, n_atoms, nf, nd):
    nwin = nwin_tot - 1

    def wmapf(u, wi, bi, li):
        return (jnp.minimum(wi[u], nwin - 1), 0)

    def wmap3(u, wi, bi, li):
        return (wi[u], 0, 0)

    def bmap(u, wi, bi, li):
        return (bi[u], 0)

    const2 = lambda u, wi, bi, li: (0, 0)
    return pltpu.PrefetchScalarGridSpec(
        num_scalar_prefetch=3,
        grid=(U,),
        in_specs=[
            pl.BlockSpec((W, nf), wmapf),
            pl.BlockSpec((1, 1, W), wmap3),
            pl.BlockSpec((1, 8, W), wmap3),
            pl.BlockSpec((nd * nf, nf), const2),
            pl.BlockSpec((nf, nf), const2),
            pl.BlockSpec((32, 128), const2),
            pl.BlockSpec((TA, nf), bmap),
        ],
        out_specs=pl.BlockSpec((TA, nf), bmap),
        scratch_shapes=[pltpu.VMEM((TA, 4 * nf), jnp.float32),
                        pltpu.VMEM((W, 4 * nf), jnp.bfloat16)],
    )


def kernel(in_features, pair_first, pair_second, dist_pairs, coord_pairs,
           int_weights, self_W, self_b, vecscales, mu, sigma):
    n_atoms, nf = in_features.shape
    E = pair_first.shape[0]
    nd = mu.shape[0]
    nwin = E // W
    nwin_tot = nwin + 1
    nblk = n_atoms // TA
    U = nwin + nblk

    feat_g = _sc_gather(in_features, pair_second.astype(jnp.int32))

    pf_pad = jnp.concatenate(
        [pair_first, jnp.full((W,), n_atoms, jnp.int32)])
    ed = jnp.concatenate(
        [dist_pairs[:, None], coord_pairs, jnp.zeros((E, 4), jnp.float32)],
        axis=1)
    ed_tail = jnp.tile(jnp.array([[1.0] + [0.0] * 7], jnp.float32), (W, 1))
    ed = jnp.concatenate([ed, ed_tail], axis=0).reshape(
        nwin_tot, W, 8).transpose(0, 2, 1)
    pf3 = pf_pad.reshape(nwin_tot, 1, W)

    fb = pair_first[::W] // TA
    lb = pair_first[W - 1::W] // TA
    cs = jnp.concatenate(
        [jnp.zeros((1,), lb.dtype), jnp.minimum(fb[1:], lb[:-1] + 1)])
    ce = jnp.concatenate([lb[:-1], jnp.full((1,), nblk - 1, lb.dtype)])
    counts = ce - cs + 1
    off = jnp.concatenate(
        [jnp.zeros((1,), jnp.int32), jnp.cumsum(counts).astype(jnp.int32)])
    u_real = off[-1]
    uu = jnp.arange(U, dtype=jnp.int32)
    w_of = jnp.clip(jnp.searchsorted(off, uu, side="right") - 1, 0, nwin - 1)
    b_of = cs[w_of].astype(jnp.int32) + (uu - off[w_of])
    valid = uu < u_real
    win_ids = jnp.where(valid, w_of, nwin).astype(jnp.int32)
    blk_ids = jnp.where(valid, b_of, nblk - 1).astype(jnp.int32)
    live_ids = valid.astype(jnp.int32)

    wrs = jnp.transpose(int_weights, (0, 2, 1)).reshape(
        nd * nf, -1).astype(jnp.bfloat16)
    sw_t = self_W.T
    params = jnp.zeros((32, 128), jnp.float32)
    params = params.at[0, :].set(vecscales)
    params = params.at[1, :].set(self_b)
    params = params.at[8:8 + nd, 0].set(mu)
    params = params.at[8:8 + nd, 1].set(1.0 / sigma)

    out = pl.pallas_call(
        _tc_body,
        grid_spec=_make_grid_spec(U, nwin_tot, n_atoms, nf, nd),
        out_shape=jax.ShapeDtypeStruct((n_atoms, nf), jnp.float32),
        compiler_params=pltpu.CompilerParams(
            dimension_semantics=("arbitrary",)),
    )(win_ids, blk_ids, live_ids, feat_g, pf3, ed, wrs, sw_t, params,
      in_features)
    return out


# speedup vs baseline: 75.4634x; 1.0963x over previous
import functools

import jax
import jax.numpy as jnp
from jax import lax
from jax.experimental import pallas as pl
from jax.experimental.pallas import tpu as pltpu
from jax.experimental.pallas import tpu_sc as plsc

HARD_CUTOFF = 5.5
CUSP_REG = 1e-06
W = 1280
TA = 80


def _sc_gather(table, idx):
    E = idx.shape[0]
    nf = table.shape[1]
    info = plsc.get_sparse_core_info()
    nw = info.num_cores * info.num_subcores
    b_per_w = E // nw
    C = 1000
    n_iter = b_per_w // C
    mesh = plsc.VectorSubcoreMesh(core_axis_name="c", subcore_axis_name="s")

    @functools.partial(
        pl.kernel, mesh=mesh,
        out_type=jax.ShapeDtypeStruct((E, nf), jnp.float32),
        scratch_types=[
            pltpu.VMEM((C,), jnp.int32),
            pltpu.VMEM((C, nf), jnp.float32),
            pltpu.SemaphoreType.DMA,
        ],
    )
    def k(table_hbm, idx_hbm, out_hbm, idx_v, rows_v, sem):
        wid = lax.axis_index("s") * info.num_cores + lax.axis_index("c")
        base = wid * b_per_w

        def body(j, carry):
            off = base + j * C
            pltpu.sync_copy(idx_hbm.at[pl.ds(off, C)], idx_v)
            pltpu.async_copy(table_hbm.at[idx_v], rows_v, sem).wait()
            pltpu.sync_copy(rows_v, out_hbm.at[pl.ds(off, C)])
            return carry

        lax.fori_loop(0, n_iter, body, 0)

    return k(table, idx)


def _tc_body(win_ref, blk_ref, live_ref, feat_ref, pf_ref, ed_ref, wrs_ref,
             sw_ref, par_ref, inf_ref, out_ref, acc_ref, r4_ref):
    u = pl.program_id(0)
    nu = pl.num_programs(0)
    b = blk_ref[u]
    prev_b = blk_ref[jnp.maximum(u - 1, 0)]
    next_b = blk_ref[jnp.minimum(u + 1, nu - 1)]
    first = jnp.logical_or(u == 0, b != prev_b)
    last = jnp.logical_or(u == nu - 1, b != next_b)
    live = live_ref[u] > 0
    new_win = jnp.logical_or(
        u == 0, win_ref[u] != win_ref[jnp.maximum(u - 1, 0)])

    @pl.when(first)
    def _():
        acc_ref[...] = jnp.zeros_like(acc_ref)

    @pl.when(jnp.logical_and(live, new_win))
    def _():
        nf = feat_ref.shape[1]
        nd = wrs_ref.shape[0] // nf
        edt = ed_ref[0]
        dr = edt[0:1, :]
        invr = 1.0 / dr
        cutr = jnp.where(dr < HARD_CUTOFF,
                         jnp.cos((0.5 * jnp.pi / HARD_CUTOFF) * dr) ** 2, 0.0)
        ur = edt[1:4, :] * invr
        muc = par_ref[8:8 + nd, 0:1]
        isg = par_ref[8:8 + nd, 1:2]
        st = jnp.exp(-0.5 * ((invr - muc) * isg) ** 2) * cutr
        sense = st.T.astype(jnp.bfloat16)

        feat = feat_ref[...].astype(jnp.bfloat16)
        z = jnp.concatenate(
            [sense[:, c:c + 1] * feat for c in range(nd)], axis=1)
        r = jnp.dot(z, wrs_ref[...],
                    preferred_element_type=jnp.float32)
        rb = r.astype(jnp.bfloat16)

        scal = jnp.concatenate(
            [ur, jnp.zeros((5, W), jnp.float32)], axis=0)
        cols = scal.T.astype(jnp.bfloat16)
        r4_ref[:, 0:nf] = rb
        r4_ref[:, nf:2 * nf] = cols[:, 0:1] * rb
        r4_ref[:, 2 * nf:3 * nf] = cols[:, 1:2] * rb
        r4_ref[:, 3 * nf:4 * nf] = cols[:, 2:3] * rb

    @pl.when(live)
    def _():
        pf = pf_ref[0, 0, :]
        aid = b * TA + lax.broadcasted_iota(jnp.int32, (TA, W), 0)
        oh = (aid == pf[None, :]).astype(jnp.bfloat16)
        acc_ref[...] += jnp.dot(oh, r4_ref[...],
                                preferred_element_type=jnp.float32)

    @pl.when(last)
    def _():
        a = acc_ref[...]
        nf = out_ref.shape[1]
        fs = a[:, 0:nf]
        fx = a[:, nf:2 * nf]
        fy = a[:, 2 * nf:3 * nf]
        fz = a[:, 3 * nf:4 * nf]
        fv = jnp.sqrt(fx * fx + fy * fy + fz * fz + CUSP_REG) * par_ref[0:1, :]
        sp = jnp.dot(inf_ref[...], sw_ref[...],
                     preferred_element_type=jnp.float32) + par_ref[1:2, :]
        out_ref[...] = fs + fv + sp


def _make_grid_spec(U, nwin_tot, n_atoms, nf, nd):
    nwin = nwin_tot - 1

    def wmapf(u, wi, bi, li):
        return (jnp.minimum(wi[u], nwin - 1), 0)

    def wmap3(u, wi, bi, li):
        return (wi[u], 0, 0)

    def bmap(u, wi, bi, li):
        return (bi[u], 0)

    const2 = lambda u, wi, bi, li: (0, 0)
    return pltpu.PrefetchScalarGridSpec(
        num_scalar_prefetch=3,
        grid=(U,),
        in_specs=[
            pl.BlockSpec((W, nf), wmapf),
            pl.BlockSpec((1, 1, W), wmap3),
            pl.BlockSpec((1, 8, W), wmap3),
            pl.BlockSpec((nd * nf, nf), const2),
            pl.BlockSpec((nf, nf), const2),
            pl.BlockSpec((32, 128), const2),
            pl.BlockSpec((TA, nf), bmap),
        ],
        out_specs=pl.BlockSpec((TA, nf), bmap),
        scratch_shapes=[pltpu.VMEM((TA, 4 * nf), jnp.float32),
                        pltpu.VMEM((W, 4 * nf), jnp.bfloat16)],
    )


def kernel(in_features, pair_first, pair_second, dist_pairs, coord_pairs,
           int_weights, self_W, self_b, vecscales, mu, sigma):
    n_atoms, nf = in_features.shape
    E = pair_first.shape[0]
    nd = mu.shape[0]
    nwin = E // W
    nwin_tot = nwin + 1
    nblk = n_atoms // TA
    U = nwin + nblk

    feat_g = _sc_gather(in_features, pair_second.astype(jnp.int32))

    pf_pad = jnp.concatenate(
        [pair_first, jnp.full((W,), n_atoms, jnp.int32)])
    ed = jnp.concatenate(
        [dist_pairs[:, None], coord_pairs, jnp.zeros((E, 4), jnp.float32)],
        axis=1)
    ed_tail = jnp.tile(jnp.array([[1.0] + [0.0] * 7], jnp.float32), (W, 1))
    ed = jnp.concatenate([ed, ed_tail], axis=0).reshape(
        nwin_tot, W, 8).transpose(0, 2, 1)
    pf3 = pf_pad.reshape(nwin_tot, 1, W)

    fb = pair_first[::W] // TA
    lb = pair_first[W - 1::W] // TA
    cs = jnp.concatenate(
        [jnp.zeros((1,), lb.dtype), jnp.minimum(fb[1:], lb[:-1] + 1)])
    ce = jnp.concatenate([lb[:-1], jnp.full((1,), nblk - 1, lb.dtype)])
    counts = ce - cs + 1
    off = jnp.concatenate(
        [jnp.zeros((1,), jnp.int32), jnp.cumsum(counts).astype(jnp.int32)])
    u_real = off[-1]
    uu = jnp.arange(U, dtype=jnp.int32)
    w_of = jnp.clip(jnp.searchsorted(off, uu, side="right") - 1, 0, nwin - 1)
    b_of = cs[w_of].astype(jnp.int32) + (uu - off[w_of])
    valid = uu < u_real
    win_ids = jnp.where(valid, w_of, nwin).astype(jnp.int32)
    blk_ids = jnp.where(valid, b_of, nblk - 1).astype(jnp.int32)
    live_ids = valid.astype(jnp.int32)

    wrs = jnp.transpose(int_weights, (0, 2, 1)).reshape(
        nd * nf, -1).astype(jnp.bfloat16)
    sw_t = self_W.T
    params = jnp.zeros((32, 128), jnp.float32)
    params = params.at[0, :].set(vecscales)
    params = params.at[1, :].set(self_b)
    params = params.at[8:8 + nd, 0].set(mu)
    params = params.at[8:8 + nd, 1].set(1.0 / sigma)

    out = pl.pallas_call(
        _tc_body,
        grid_spec=_make_grid_spec(U, nwin_tot, n_atoms, nf, nd),
        out_shape=jax.ShapeDtypeStruct((n_atoms, nf), jnp.float32),
        compiler_params=pltpu.CompilerParams(
            dimension_semantics=("arbitrary",)),
    )(win_ids, blk_ids, live_ids, feat_g, pf3, ed, wrs, sw_t, params,
      in_features)
    return out


# speedup vs baseline: 84.5561x; 1.1205x over previous
import functools

import jax
import jax.numpy as jnp
from jax import lax
from jax.experimental import pallas as pl
from jax.experimental.pallas import tpu as pltpu
from jax.experimental.pallas import tpu_sc as plsc

HARD_CUTOFF = 5.5
CUSP_REG = 1e-06
W = 1280
TA = 200


def _sc_gather(table, idx):
    E = idx.shape[0]
    nf = table.shape[1]
    info = plsc.get_sparse_core_info()
    nw = info.num_cores * info.num_subcores
    b_per_w = E // nw
    C = 1000
    n_iter = b_per_w // C
    mesh = plsc.VectorSubcoreMesh(core_axis_name="c", subcore_axis_name="s")

    @functools.partial(
        pl.kernel, mesh=mesh,
        out_type=jax.ShapeDtypeStruct((E, nf), jnp.float32),
        scratch_types=[
            pltpu.VMEM((C,), jnp.int32),
            pltpu.VMEM((C, nf), jnp.float32),
            pltpu.SemaphoreType.DMA,
        ],
    )
    def k(table_hbm, idx_hbm, out_hbm, idx_v, rows_v, sem):
        wid = lax.axis_index("s") * info.num_cores + lax.axis_index("c")
        base = wid * b_per_w

        def body(j, carry):
            off = base + j * C
            pltpu.sync_copy(idx_hbm.at[pl.ds(off, C)], idx_v)
            pltpu.async_copy(table_hbm.at[idx_v], rows_v, sem).wait()
            pltpu.sync_copy(rows_v, out_hbm.at[pl.ds(off, C)])
            return carry

        lax.fori_loop(0, n_iter, body, 0)

    return k(table, idx)


def _tc_body(win_ref, blk_ref, live_ref, feat_ref, pf_ref, ed_ref, wrs_ref,
             sw_ref, par_ref, inf_ref, out_ref, acc_ref, r4_ref):
    u = pl.program_id(0)
    nu = pl.num_programs(0)
    b = blk_ref[u]
    prev_b = blk_ref[jnp.maximum(u - 1, 0)]
    next_b = blk_ref[jnp.minimum(u + 1, nu - 1)]
    first = jnp.logical_or(u == 0, b != prev_b)
    last = jnp.logical_or(u == nu - 1, b != next_b)
    live = live_ref[u] > 0
    new_win = jnp.logical_or(
        u == 0, win_ref[u] != win_ref[jnp.maximum(u - 1, 0)])

    @pl.when(first)
    def _():
        acc_ref[...] = jnp.zeros_like(acc_ref)

    @pl.when(jnp.logical_and(live, new_win))
    def _():
        nf = feat_ref.shape[1]
        nd = wrs_ref.shape[0] // nf
        edt = ed_ref[0]
        dr = edt[0:1, :]
        invr = 1.0 / dr
        cutr = jnp.where(dr < HARD_CUTOFF,
                         jnp.cos((0.5 * jnp.pi / HARD_CUTOFF) * dr) ** 2, 0.0)
        ur = edt[1:4, :] * invr
        muc = par_ref[8:8 + nd, 0:1]
        isg = par_ref[8:8 + nd, 1:2]
        st = jnp.exp(-0.5 * ((invr - muc) * isg) ** 2) * cutr
        sense = st.T.astype(jnp.bfloat16)

        feat = feat_ref[...].astype(jnp.bfloat16)
        z = jnp.concatenate(
            [sense[:, c:c + 1] * feat for c in range(nd)], axis=1)
        r = jnp.dot(z, wrs_ref[...],
                    preferred_element_type=jnp.float32)
        rb = r.astype(jnp.bfloat16)

        scal = jnp.concatenate(
            [ur, jnp.zeros((5, W), jnp.float32)], axis=0)
        cols = scal.T.astype(jnp.bfloat16)
        r4_ref[:, 0:nf] = rb
        r4_ref[:, nf:2 * nf] = cols[:, 0:1] * rb
        r4_ref[:, 2 * nf:3 * nf] = cols[:, 1:2] * rb
        r4_ref[:, 3 * nf:4 * nf] = cols[:, 2:3] * rb

    @pl.when(live)
    def _():
        pf = pf_ref[0, 0, :]
        aid = b * TA + lax.broadcasted_iota(jnp.int32, (TA, W), 0)
        oh = (aid == pf[None, :]).astype(jnp.bfloat16)
        acc_ref[...] += jnp.dot(oh, r4_ref[...],
                                preferred_element_type=jnp.float32)

    @pl.when(last)
    def _():
        a = acc_ref[...]
        nf = out_ref.shape[1]
        fs = a[:, 0:nf]
        fx = a[:, nf:2 * nf]
        fy = a[:, 2 * nf:3 * nf]
        fz = a[:, 3 * nf:4 * nf]
        fv = jnp.sqrt(fx * fx + fy * fy + fz * fz + CUSP_REG) * par_ref[0:1, :]
        sp = jnp.dot(inf_ref[...], sw_ref[...],
                     preferred_element_type=jnp.float32) + par_ref[1:2, :]
        out_ref[...] = fs + fv + sp


def _make_grid_spec(U, nwin_tot, n_atoms, nf, nd):
    nwin = nwin_tot - 1

    def wmapf(u, wi, bi, li):
        return (jnp.minimum(wi[u], nwin - 1), 0)

    def wmap3(u, wi, bi, li):
        return (wi[u], 0, 0)

    def bmap(u, wi, bi, li):
        return (bi[u], 0)

    const2 = lambda u, wi, bi, li: (0, 0)
    return pltpu.PrefetchScalarGridSpec(
        num_scalar_prefetch=3,
        grid=(U,),
        in_specs=[
            pl.BlockSpec((W, nf), wmapf),
            pl.BlockSpec((1, 1, W), wmap3),
            pl.BlockSpec((1, 8, W), wmap3),
            pl.BlockSpec((nd * nf, nf), const2),
            pl.BlockSpec((nf, nf), const2),
            pl.BlockSpec((32, 128), const2),
            pl.BlockSpec((TA, nf), bmap),
        ],
        out_specs=pl.BlockSpec((TA, nf), bmap),
        scratch_shapes=[pltpu.VMEM((TA, 4 * nf), jnp.float32),
                        pltpu.VMEM((W, 4 * nf), jnp.bfloat16)],
    )


def kernel(in_features, pair_first, pair_second, dist_pairs, coord_pairs,
           int_weights, self_W, self_b, vecscales, mu, sigma):
    n_atoms, nf = in_features.shape
    E = pair_first.shape[0]
    nd = mu.shape[0]
    nwin = E // W
    nwin_tot = nwin + 1
    nblk = n_atoms // TA
    U = nwin + nblk

    feat_g = _sc_gather(in_features, pair_second.astype(jnp.int32))

    pf_pad = jnp.concatenate(
        [pair_first, jnp.full((W,), n_atoms, jnp.int32)])
    ed = jnp.concatenate(
        [dist_pairs[:, None], coord_pairs, jnp.zeros((E, 4), jnp.float32)],
        axis=1)
    ed_tail = jnp.tile(jnp.array([[1.0] + [0.0] * 7], jnp.float32), (W, 1))
    ed = jnp.concatenate([ed, ed_tail], axis=0).reshape(
        nwin_tot, W, 8).transpose(0, 2, 1)
    pf3 = pf_pad.reshape(nwin_tot, 1, W)

    fb = pair_first[::W] // TA
    lb = pair_first[W - 1::W] // TA
    cs = jnp.concatenate(
        [jnp.zeros((1,), lb.dtype), jnp.minimum(fb[1:], lb[:-1] + 1)])
    ce = jnp.concatenate([lb[:-1], jnp.full((1,), nblk - 1, lb.dtype)])
    counts = ce - cs + 1
    off = jnp.concatenate(
        [jnp.zeros((1,), jnp.int32), jnp.cumsum(counts).astype(jnp.int32)])
    u_real = off[-1]
    uu = jnp.arange(U, dtype=jnp.int32)
    w_of = jnp.clip(jnp.searchsorted(off, uu, side="right") - 1, 0, nwin - 1)
    b_of = cs[w_of].astype(jnp.int32) + (uu - off[w_of])
    valid = uu < u_real
    win_ids = jnp.where(valid, w_of, nwin).astype(jnp.int32)
    blk_ids = jnp.where(valid, b_of, nblk - 1).astype(jnp.int32)
    live_ids = valid.astype(jnp.int32)

    wrs = jnp.transpose(int_weights, (0, 2, 1)).reshape(
        nd * nf, -1).astype(jnp.bfloat16)
    sw_t = self_W.T
    params = jnp.zeros((32, 128), jnp.float32)
    params = params.at[0, :].set(vecscales)
    params = params.at[1, :].set(self_b)
    params = params.at[8:8 + nd, 0].set(mu)
    params = params.at[8:8 + nd, 1].set(1.0 / sigma)

    out = pl.pallas_call(
        _tc_body,
        grid_spec=_make_grid_spec(U, nwin_tot, n_atoms, nf, nd),
        out_shape=jax.ShapeDtypeStruct((n_atoms, nf), jnp.float32),
        compiler_params=pltpu.CompilerParams(
            dimension_semantics=("arbitrary",)),
    )(win_ids, blk_ids, live_ids, feat_g, pf3, ed, wrs, sw_t, params,
      in_features)
    return out


# speedup vs baseline: 87.9600x; 1.0403x over previous
import functools

import jax
import jax.numpy as jnp
from jax import lax
from jax.experimental import pallas as pl
from jax.experimental.pallas import tpu as pltpu
from jax.experimental.pallas import tpu_sc as plsc

HARD_CUTOFF = 5.5
CUSP_REG = 1e-06
W = 2000
TA = 200


def _sc_gather(table, idx):
    E = idx.shape[0]
    nf = table.shape[1]
    info = plsc.get_sparse_core_info()
    nw = info.num_cores * info.num_subcores
    b_per_w = E // nw
    C = 1000
    n_iter = b_per_w // C
    mesh = plsc.VectorSubcoreMesh(core_axis_name="c", subcore_axis_name="s")

    @functools.partial(
        pl.kernel, mesh=mesh,
        out_type=jax.ShapeDtypeStruct((E, nf), jnp.float32),
        scratch_types=[
            pltpu.VMEM((C,), jnp.int32),
            pltpu.VMEM((C, nf), jnp.float32),
            pltpu.SemaphoreType.DMA,
        ],
    )
    def k(table_hbm, idx_hbm, out_hbm, idx_v, rows_v, sem):
        wid = lax.axis_index("s") * info.num_cores + lax.axis_index("c")
        base = wid * b_per_w

        def body(j, carry):
            off = base + j * C
            pltpu.sync_copy(idx_hbm.at[pl.ds(off, C)], idx_v)
            pltpu.async_copy(table_hbm.at[idx_v], rows_v, sem).wait()
            pltpu.sync_copy(rows_v, out_hbm.at[pl.ds(off, C)])
            return carry

        lax.fori_loop(0, n_iter, body, 0)

    return k(table, idx)


def _tc_body(win_ref, blk_ref, live_ref, feat_ref, pf_ref, ed_ref, wrs_ref,
             sw_ref, par_ref, inf_ref, out_ref, acc_ref, r4_ref):
    u = pl.program_id(0)
    nu = pl.num_programs(0)
    b = blk_ref[u]
    prev_b = blk_ref[jnp.maximum(u - 1, 0)]
    next_b = blk_ref[jnp.minimum(u + 1, nu - 1)]
    first = jnp.logical_or(u == 0, b != prev_b)
    last = jnp.logical_or(u == nu - 1, b != next_b)
    live = live_ref[u] > 0
    new_win = jnp.logical_or(
        u == 0, win_ref[u] != win_ref[jnp.maximum(u - 1, 0)])

    @pl.when(first)
    def _():
        acc_ref[...] = jnp.zeros_like(acc_ref)

    @pl.when(jnp.logical_and(live, new_win))
    def _():
        nf = feat_ref.shape[1]
        nd = wrs_ref.shape[0] // nf
        edt = ed_ref[0]
        dr = edt[0:1, :]
        invr = 1.0 / dr
        cutr = jnp.where(dr < HARD_CUTOFF,
                         jnp.cos((0.5 * jnp.pi / HARD_CUTOFF) * dr) ** 2, 0.0)
        ur = edt[1:4, :] * invr
        muc = par_ref[8:8 + nd, 0:1]
        isg = par_ref[8:8 + nd, 1:2]
        st = jnp.exp(-0.5 * ((invr - muc) * isg) ** 2) * cutr
        sense = st.T.astype(jnp.bfloat16)

        feat = feat_ref[...].astype(jnp.bfloat16)
        z = jnp.concatenate(
            [sense[:, c:c + 1] * feat for c in range(nd)], axis=1)
        r = jnp.dot(z, wrs_ref[...],
                    preferred_element_type=jnp.float32)
        rb = r.astype(jnp.bfloat16)

        scal = jnp.concatenate(
            [ur, jnp.zeros((5, W), jnp.float32)], axis=0)
        cols = scal.T.astype(jnp.bfloat16)
        r4_ref[:, 0:nf] = rb
        r4_ref[:, nf:2 * nf] = cols[:, 0:1] * rb
        r4_ref[:, 2 * nf:3 * nf] = cols[:, 1:2] * rb
        r4_ref[:, 3 * nf:4 * nf] = cols[:, 2:3] * rb

    @pl.when(live)
    def _():
        pf = pf_ref[0, 0, :]
        aid = b * TA + lax.broadcasted_iota(jnp.int32, (TA, W), 0)
        oh = (aid == pf[None, :]).astype(jnp.bfloat16)
        acc_ref[...] += jnp.dot(oh, r4_ref[...],
                                preferred_element_type=jnp.float32)

    @pl.when(last)
    def _():
        a = acc_ref[...]
        nf = out_ref.shape[1]
        fs = a[:, 0:nf]
        fx = a[:, nf:2 * nf]
        fy = a[:, 2 * nf:3 * nf]
        fz = a[:, 3 * nf:4 * nf]
        fv = jnp.sqrt(fx * fx + fy * fy + fz * fz + CUSP_REG) * par_ref[0:1, :]
        sp = jnp.dot(inf_ref[...], sw_ref[...],
                     preferred_element_type=jnp.float32) + par_ref[1:2, :]
        out_ref[...] = fs + fv + sp


def _make_grid_spec(U, nwin_tot, n_atoms, nf, nd):
    nwin = nwin_tot - 1

    def wmapf(u, wi, bi, li):
        return (jnp.minimum(wi[u], nwin - 1), 0)

    def wmap3(u, wi, bi, li):
        return (wi[u], 0, 0)

    def bmap(u, wi, bi, li):
        return (bi[u], 0)

    const2 = lambda u, wi, bi, li: (0, 0)
    return pltpu.PrefetchScalarGridSpec(
        num_scalar_prefetch=3,
        grid=(U,),
        in_specs=[
            pl.BlockSpec((W, nf), wmapf),
            pl.BlockSpec((1, 1, W), wmap3),
            pl.BlockSpec((1, 8, W), wmap3),
            pl.BlockSpec((nd * nf, nf), const2),
            pl.BlockSpec((nf, nf), const2),
            pl.BlockSpec((32, 128), const2),
            pl.BlockSpec((TA, nf), bmap),
        ],
        out_specs=pl.BlockSpec((TA, nf), bmap),
        scratch_shapes=[pltpu.VMEM((TA, 4 * nf), jnp.float32),
                        pltpu.VMEM((W, 4 * nf), jnp.bfloat16)],
    )


def kernel(in_features, pair_first, pair_second, dist_pairs, coord_pairs,
           int_weights, self_W, self_b, vecscales, mu, sigma):
    n_atoms, nf = in_features.shape
    E = pair_first.shape[0]
    nd = mu.shape[0]
    nwin = E // W
    nwin_tot = nwin + 1
    nblk = n_atoms // TA
    U = nwin + nblk

    feat_g = _sc_gather(in_features, pair_second.astype(jnp.int32))

    pf_pad = jnp.concatenate(
        [pair_first, jnp.full((W,), n_atoms, jnp.int32)])
    ed = jnp.concatenate(
        [dist_pairs[:, None], coord_pairs, jnp.zeros((E, 4), jnp.float32)],
        axis=1)
    ed_tail = jnp.tile(jnp.array([[1.0] + [0.0] * 7], jnp.float32), (W, 1))
    ed = jnp.concatenate([ed, ed_tail], axis=0).reshape(
        nwin_tot, W, 8).transpose(0, 2, 1)
    pf3 = pf_pad.reshape(nwin_tot, 1, W)

    fb = pair_first[::W] // TA
    lb = pair_first[W - 1::W] // TA
    cs = jnp.concatenate(
        [jnp.zeros((1,), lb.dtype), jnp.minimum(fb[1:], lb[:-1] + 1)])
    ce = jnp.concatenate([lb[:-1], jnp.full((1,), nblk - 1, lb.dtype)])
    counts = ce - cs + 1
    off = jnp.concatenate(
        [jnp.zeros((1,), jnp.int32), jnp.cumsum(counts).astype(jnp.int32)])
    u_real = off[-1]
    uu = jnp.arange(U, dtype=jnp.int32)
    w_of = jnp.clip(jnp.searchsorted(off, uu, side="right") - 1, 0, nwin - 1)
    b_of = cs[w_of].astype(jnp.int32) + (uu - off[w_of])
    valid = uu < u_real
    win_ids = jnp.where(valid, w_of, nwin).astype(jnp.int32)
    blk_ids = jnp.where(valid, b_of, nblk - 1).astype(jnp.int32)
    live_ids = valid.astype(jnp.int32)

    wrs = jnp.transpose(int_weights, (0, 2, 1)).reshape(
        nd * nf, -1).astype(jnp.bfloat16)
    sw_t = self_W.T
    params = jnp.zeros((32, 128), jnp.float32)
    params = params.at[0, :].set(vecscales)
    params = params.at[1, :].set(self_b)
    params = params.at[8:8 + nd, 0].set(mu)
    params = params.at[8:8 + nd, 1].set(1.0 / sigma)

    out = pl.pallas_call(
        _tc_body,
        grid_spec=_make_grid_spec(U, nwin_tot, n_atoms, nf, nd),
        out_shape=jax.ShapeDtypeStruct((n_atoms, nf), jnp.float32),
        compiler_params=pltpu.CompilerParams(
            dimension_semantics=("arbitrary",)),
    )(win_ids, blk_ids, live_ids, feat_g, pf3, ed, wrs, sw_t, params,
      in_features)
    return out
